# baseline (speedup 1.0000x reference)
import functools

import jax
import jax.numpy as jnp
from jax import lax
from jax.experimental import pallas as pl
from jax.experimental.pallas import tpu as pltpu
from jax.experimental.pallas import tpu_sc as plsc

VOCAB = 1000000
EMBED = 64

NC = 2
NS = 16
NW = NC * NS

CHUNK = 128


def _sc_gather_pairs(table, gidx, n_rows):
    b_per_w = n_rows // NW
    n_chunks = b_per_w // CHUNK
    mesh = plsc.VectorSubcoreMesh(core_axis_name="c", subcore_axis_name="s")

    @functools.partial(
        pl.kernel,
        out_type=jax.ShapeDtypeStruct((n_rows, 2 * EMBED), jnp.float32),
        mesh=mesh,
        scratch_types=[
            pltpu.VMEM((b_per_w,), jnp.int32),
            pltpu.VMEM((CHUNK, 2 * EMBED), jnp.float32),
            pltpu.SemaphoreType.DMA,
        ],
    )
    def k(table_hbm, gidx_hbm, out_hbm, idx_v, rows_v, sem):
        wid = lax.axis_index("s") * NC + lax.axis_index("c")
        base = wid * b_per_w
        pltpu.sync_copy(gidx_hbm.at[pl.ds(base, b_per_w)], idx_v)

        @pl.loop(0, n_chunks)
        def _chunk(c):
            off = c * CHUNK
            pltpu.async_copy(
                table_hbm.at[idx_v.at[pl.ds(off, CHUNK)]], rows_v, sem
            ).wait()
            pltpu.sync_copy(rows_v, out_hbm.at[pl.ds(base + off, CHUNK)])

    return k(table, gidx)


def kernel(x, w):
    b, l = x.shape
    n = b * l
    idx = x.reshape(-1).astype(jnp.int32)
    table = jnp.transpose(w).reshape(VOCAB // 2, 2 * EMBED)
    pairs = _sc_gather_pairs(table, idx >> 1, n)
    odd = (idx & 1)[:, None] == 1
    out = jnp.where(odd, pairs[:, EMBED:], pairs[:, :EMBED])
    return out.reshape(b, l, EMBED)


# speedup vs baseline: 1.0288x; 1.0288x over previous
import functools

import jax
import jax.numpy as jnp
from jax import lax
from jax.experimental import pallas as pl
from jax.experimental.pallas import tpu as pltpu
from jax.experimental.pallas import tpu_sc as plsc

VOCAB = 1000000
EMBED = 64

NC = 2
NS = 16
NW = NC * NS

CHUNK = 128


def _sc_gather_pairs(table, gidx, n_rows):
    b_per_w = n_rows // NW
    n_chunks = b_per_w // CHUNK
    mesh = plsc.VectorSubcoreMesh(core_axis_name="c", subcore_axis_name="s")

    @functools.partial(
        pl.kernel,
        out_type=jax.ShapeDtypeStruct((n_rows, 2 * EMBED), jnp.float32),
        mesh=mesh,
        scratch_types=[
            pltpu.VMEM((b_per_w,), jnp.int32),
            pltpu.VMEM((CHUNK, 2 * EMBED), jnp.float32),
            pltpu.SemaphoreType.DMA,
        ],
        compiler_params=pltpu.CompilerParams(use_tc_tiling_on_sc=True),
    )
    def k(table_hbm, gidx_hbm, out_hbm, idx_v, rows_v, sem):
        wid = lax.axis_index("s") * NC + lax.axis_index("c")
        base = wid * b_per_w
        pltpu.sync_copy(gidx_hbm.at[pl.ds(base, b_per_w)], idx_v)

        @pl.loop(0, n_chunks)
        def _chunk(c):
            off = c * CHUNK
            pltpu.async_copy(
                table_hbm.at[idx_v.at[pl.ds(off, CHUNK)]], rows_v, sem
            ).wait()
            pltpu.sync_copy(rows_v, out_hbm.at[pl.ds(base + off, CHUNK)])

    return k(table, gidx)


def kernel(x, w):
    b, l = x.shape
    n = b * l
    idx = x.reshape(-1).astype(jnp.int32)
    table = jnp.transpose(w).reshape(VOCAB // 2, 2 * EMBED)
    pairs = _sc_gather_pairs(table, idx >> 1, n)
    p = (idx & 1).astype(jnp.float32)[:, None]
    out = pairs[:, :EMBED] * (1.0 - p) + pairs[:, EMBED:] * p
    return out.reshape(b, l, EMBED)


# speedup vs baseline: 1.2465x; 1.2116x over previous
import functools

import jax
import jax.numpy as jnp
from jax import lax
from jax.experimental import pallas as pl
from jax.experimental.pallas import tpu as pltpu
from jax.experimental.pallas import tpu_sc as plsc

VOCAB = 1000000
EMBED = 64
ROW = 128

NC = 2
NS = 16
NW = NC * NS

CHUNK = 128


def _sc_gather(table, gidx, n_rows):
    b_per_w = n_rows // NW
    n_chunks = b_per_w // CHUNK
    mesh = plsc.VectorSubcoreMesh(core_axis_name="c", subcore_axis_name="s")

    @functools.partial(
        pl.kernel,
        out_type=jax.ShapeDtypeStruct((n_rows, ROW), jnp.float32),
        mesh=mesh,
        scratch_types=[
            pltpu.VMEM((b_per_w,), jnp.int32),
            pltpu.VMEM((CHUNK, ROW), jnp.float32),
            pltpu.SemaphoreType.DMA,
        ],
        compiler_params=pltpu.CompilerParams(use_tc_tiling_on_sc=True),
    )
    def k(table_hbm, gidx_hbm, out_hbm, idx_v, rows_v, sem):
        wid = lax.axis_index("s") * NC + lax.axis_index("c")
        base = wid * b_per_w
        pltpu.sync_copy(gidx_hbm.at[pl.ds(base, b_per_w)], idx_v)

        @pl.loop(0, n_chunks)
        def _chunk(c):
            off = c * CHUNK
            pltpu.async_copy(
                table_hbm.at[idx_v.at[pl.ds(off, CHUNK)]], rows_v, sem
            ).wait()
            pltpu.sync_copy(rows_v, out_hbm.at[pl.ds(base + off, CHUNK)])

    return k(table, gidx)


def kernel(x, w):
    b, l = x.shape
    n = b * l
    idx = x.reshape(-1).astype(jnp.int32)
    wp = jnp.pad(w, ((0, ROW - EMBED), (0, 0)))
    table = jnp.transpose(wp)
    rows = _sc_gather(table, idx, n)
    return rows.reshape(b, l, ROW)[:, :, :EMBED]


# speedup vs baseline: 1.3032x; 1.0455x over previous
import functools

import jax
import jax.numpy as jnp
from jax import lax
from jax.experimental import pallas as pl
from jax.experimental.pallas import tpu as pltpu
from jax.experimental.pallas import tpu_sc as plsc

VOCAB = 1000000
EMBED = 64
ROW = 128

NC = 2
NS = 16
NW = NC * NS

CHUNK = 128


def _sc_gather(table, gidx, n_rows):
    b_per_w = n_rows // NW
    n_pairs = b_per_w // (2 * CHUNK)

    @functools.partial(
        pl.kernel,
        out_type=jax.ShapeDtypeStruct((n_rows, ROW), jnp.float32),
        mesh=plsc.VectorSubcoreMesh(core_axis_name="c", subcore_axis_name="s"),
        scratch_types=[
            pltpu.VMEM((b_per_w,), jnp.int32),
            pltpu.VMEM((CHUNK, ROW), jnp.float32),
            pltpu.VMEM((CHUNK, ROW), jnp.float32),
            pltpu.SemaphoreType.DMA,
            pltpu.SemaphoreType.DMA,
        ],
        compiler_params=pltpu.CompilerParams(use_tc_tiling_on_sc=True),
    )
    def k(table_hbm, gidx_hbm, out_hbm, idx_v, buf0, buf1, sem0, sem1):
        wid = lax.axis_index("s") * NC + lax.axis_index("c")
        base = wid * b_per_w
        pltpu.sync_copy(gidx_hbm.at[pl.ds(base, b_per_w)], idx_v)

        def start(c, buf, sem):
            pltpu.async_copy(
                table_hbm.at[idx_v.at[pl.ds(c * CHUNK, CHUNK)]], buf, sem
            )

        def drain(c, buf, sem):
            pltpu.make_async_copy(
                table_hbm.at[idx_v.at[pl.ds(0, CHUNK)]], buf, sem
            ).wait()
            pltpu.sync_copy(buf, out_hbm.at[pl.ds(base + c * CHUNK, CHUNK)])

        start(0, buf0, sem0)

        @pl.loop(0, n_pairs)
        def _pair(i):
            c0 = 2 * i
            start(c0 + 1, buf1, sem1)
            drain(c0, buf0, sem0)

            @pl.when(i < n_pairs - 1)
            def _():
                start(c0 + 2, buf0, sem0)

            drain(c0 + 1, buf1, sem1)

    return k(table, gidx)


def kernel(x, w):
    b, l = x.shape
    n = b * l
    idx = x.reshape(-1).astype(jnp.int32)
    wp = jnp.pad(w, ((0, ROW - EMBED), (0, 0)))
    table = jnp.transpose(wp)
    rows = _sc_gather(table, idx, n)
    return rows.reshape(b, l, ROW)[:, :, :EMBED]
